# Initial kernel scaffold; baseline (speedup 1.0000x reference)
import jax
import jax.numpy as jnp
from jax.experimental import pallas as pl


def kernel(x, feature_grid):
    raise NotImplementedError("write your pallas kernel here")


# speedup vs baseline: 1.6744x; 1.6744x over previous
import functools

import jax
import jax.numpy as jnp
from jax import lax
from jax.experimental import pallas as pl
from jax.experimental.pallas import tpu as pltpu
from jax.experimental.pallas import tpu_sc as plsc

D = H = W = 128
F = 16
B = 1048576

NC = 2
NS = 16
NW = NC * NS
PER_W = B // NW
C = 512
NSUB = C // 128
NCHUNK = PER_W // C
L = 16


def _body(x_hbm, table_hbm, out_hbm, coords_v, idx_v, fx_v, fy_v, fz_v,
          rows_v, out_v, sem):
    cid = lax.axis_index("c")
    sid = lax.axis_index("s")
    wid = sid * NC + cid
    base_w = wid * PER_W

    def chunk_body(g, _):
        base = base_w + g * C

        pltpu.sync_copy(x_hbm.at[pl.ds(base * 3, C * 3)], coords_v)

        def grp(j, _):
            pts3 = (j * L + lax.iota(jnp.int32, L)) * 3
            xv = plsc.load_gather(coords_v, [pts3])
            yv = plsc.load_gather(coords_v, [pts3 + 1])
            zv = plsc.load_gather(coords_v, [pts3 + 2])
            cx = (xv + 1.0) * 0.5 * (W - 1)
            cy = (yv + 1.0) * 0.5 * (H - 1)
            cz = (zv + 1.0) * 0.5 * (D - 1)
            x0 = jnp.minimum(cx.astype(jnp.int32), W - 1)
            y0 = jnp.minimum(cy.astype(jnp.int32), H - 1)
            z0 = jnp.minimum(cz.astype(jnp.int32), D - 1)
            fx = cx - x0.astype(jnp.float32)
            fy = cy - y0.astype(jnp.float32)
            fz = cz - z0.astype(jnp.float32)
            dx = jnp.minimum(x0 + 1, W - 1) - x0
            dy = (jnp.minimum(y0 + 1, H - 1) - y0) * W
            dz = (jnp.minimum(z0 + 1, D - 1) - z0) * (H * W)
            k0 = (z0 * H + y0) * W + x0
            corners = (k0, k0 + dx, k0 + dy, k0 + dy + dx,
                       k0 + dz, k0 + dz + dx, k0 + dz + dy, k0 + dz + dy + dx)
            row = j // 8
            off = (j % 8) * L
            for c in range(8):
                idx_v[c * NSUB + row, pl.ds(off, L)] = corners[c]
            fx_v[pl.ds(j * L, L)] = fx
            fy_v[pl.ds(j * L, L)] = fy
            fz_v[pl.ds(j * L, L)] = fz
            return _

        lax.fori_loop(0, C // L, grp, None)

        copies = []
        for c in range(8):
            for s in range(NSUB):
                copies.append(pltpu.async_copy(
                    table_hbm.at[idx_v.at[c * NSUB + s]],
                    rows_v.at[c, pl.ds(s * 128, 128)],
                    sem,
                ))
        for cp in copies:
            cp.wait()

        def comb(j, _):
            fxg = fx_v[pl.ds(j * L, L)]
            fyg = fy_v[pl.ds(j * L, L)]
            fzg = fz_v[pl.ds(j * L, L)]
            for k in range(L):
                i = j * L + k
                fxk = lax.broadcast_in_dim(fxg[k], (L,), ())
                fyk = lax.broadcast_in_dim(fyg[k], (L,), ())
                fzk = lax.broadcast_in_dim(fzg[k], (L,), ())
                r000 = rows_v[0, i]
                r001 = rows_v[1, i]
                r010 = rows_v[2, i]
                r011 = rows_v[3, i]
                r100 = rows_v[4, i]
                r101 = rows_v[5, i]
                r110 = rows_v[6, i]
                r111 = rows_v[7, i]
                a00 = r000 + fxk * (r001 - r000)
                a01 = r010 + fxk * (r011 - r010)
                a10 = r100 + fxk * (r101 - r100)
                a11 = r110 + fxk * (r111 - r110)
                b0 = a00 + fyk * (a01 - a00)
                b1 = a10 + fyk * (a11 - a10)
                out_v[i] = b0 + fzk * (b1 - b0)
            return _

        lax.fori_loop(0, C // L, comb, None)

        pltpu.sync_copy(out_v, out_hbm.at[pl.ds(base, C)])
        return _

    lax.fori_loop(0, NCHUNK, chunk_body, None)


@functools.partial(
    pl.kernel,
    out_type=jax.ShapeDtypeStruct((B, F), jnp.float32),
    mesh=plsc.VectorSubcoreMesh(core_axis_name="c", subcore_axis_name="s"),
    compiler_params=pltpu.CompilerParams(
        needs_layout_passes=False, use_tc_tiling_on_sc=False),
    scratch_types=[
        pltpu.VMEM((C * 3,), jnp.float32),
        pltpu.VMEM((8 * NSUB, 128), jnp.int32),
        pltpu.VMEM((C,), jnp.float32),
        pltpu.VMEM((C,), jnp.float32),
        pltpu.VMEM((C,), jnp.float32),
        pltpu.VMEM((8, C, F), jnp.float32),
        pltpu.VMEM((C, F), jnp.float32),
        pltpu.SemaphoreType.DMA,
    ],
)
def _sc_interp(x_hbm, table_hbm, out_hbm, coords_v, idx_v, fx_v, fy_v, fz_v,
               rows_v, out_v, sem):
    _body(x_hbm, table_hbm, out_hbm, coords_v, idx_v, fx_v, fy_v, fz_v,
          rows_v, out_v, sem)


def kernel(x, feature_grid):
    table = jnp.transpose(feature_grid[0], (1, 2, 3, 0)).reshape(D * H * W, F)
    return _sc_interp(x.reshape(B * 3), table)


# speedup vs baseline: 1.7766x; 1.0611x over previous
import functools

import jax
import jax.numpy as jnp
from jax import lax
from jax.experimental import pallas as pl
from jax.experimental.pallas import tpu as pltpu
from jax.experimental.pallas import tpu_sc as plsc

D = H = W = 128
F = 16
B = 1048576

NC = 2
NS = 16
NW = NC * NS
PER_W = B // NW
C = 512
NSUB = C // 128
NCHUNK = PER_W // C
L = 16


DHW = D * H * W
VCHUNK = 2048
VPW = DHW // NW
NVC = VPW // VCHUNK

_SC_PARAMS = pltpu.CompilerParams(
    needs_layout_passes=False, use_tc_tiling_on_sc=False)


@functools.partial(
    pl.kernel,
    out_type=jax.ShapeDtypeStruct((DHW, F), jnp.float32),
    mesh=plsc.VectorSubcoreMesh(core_axis_name="c", subcore_axis_name="s"),
    compiler_params=_SC_PARAMS,
    scratch_types=[
        pltpu.VMEM((F * VCHUNK,), jnp.float32),
        pltpu.VMEM((VCHUNK, F), jnp.float32),
        pltpu.SemaphoreType.DMA,
    ],
)
def _sc_transpose(grid1d, table2d, blk_v, tb_v, sem):
    wid = lax.axis_index("s") * NC + lax.axis_index("c")
    v_base = wid * VPW

    def chunk(ci, _):
        v0 = v_base + ci * VCHUNK
        cps = [pltpu.async_copy(
            grid1d.at[pl.ds(f * DHW + v0, VCHUNK)],
            blk_v.at[pl.ds(f * VCHUNK, VCHUNK)], sem) for f in range(F)]
        for cp in cps:
            cp.wait()
        col = lax.iota(jnp.int32, L) * VCHUNK

        def grp(j, _):
            for k in range(L):
                v = j * L + k
                tb_v[v] = plsc.load_gather(blk_v, [col + v])
            return _

        lax.fori_loop(0, VCHUNK // L, grp, None)
        pltpu.sync_copy(tb_v, table2d.at[pl.ds(v0, VCHUNK)])
        return _

    lax.fori_loop(0, NVC, chunk, None)


def _body(x_hbm, table1d_hbm, out_hbm, coords_v, idx_v, fx_v, fy_v, fz_v,
          rows_v, out_v, sem):
    table_hbm = table1d_hbm
    cid = lax.axis_index("c")
    sid = lax.axis_index("s")
    wid = sid * NC + cid
    base_w = wid * PER_W

    def chunk_body(g, _):
        base = base_w + g * C

        pltpu.sync_copy(x_hbm.at[pl.ds(base * 3, C * 3)], coords_v)

        def grp(j, _):
            pts3 = (j * L + lax.iota(jnp.int32, L)) * 3
            xv = plsc.load_gather(coords_v, [pts3])
            yv = plsc.load_gather(coords_v, [pts3 + 1])
            zv = plsc.load_gather(coords_v, [pts3 + 2])
            cx = (xv + 1.0) * 0.5 * (W - 1)
            cy = (yv + 1.0) * 0.5 * (H - 1)
            cz = (zv + 1.0) * 0.5 * (D - 1)
            x0 = jnp.minimum(cx.astype(jnp.int32), W - 1)
            y0 = jnp.minimum(cy.astype(jnp.int32), H - 1)
            z0 = jnp.minimum(cz.astype(jnp.int32), D - 1)
            fx = cx - x0.astype(jnp.float32)
            fy = cy - y0.astype(jnp.float32)
            fz = cz - z0.astype(jnp.float32)
            dx = jnp.minimum(x0 + 1, W - 1) - x0
            dy = (jnp.minimum(y0 + 1, H - 1) - y0) * W
            dz = (jnp.minimum(z0 + 1, D - 1) - z0) * (H * W)
            k0 = (z0 * H + y0) * W + x0
            corners = (k0, k0 + dx, k0 + dy, k0 + dy + dx,
                       k0 + dz, k0 + dz + dx, k0 + dz + dy, k0 + dz + dy + dx)
            row = j // 8
            off = (j % 8) * L
            for c in range(8):
                idx_v[c * NSUB + row, pl.ds(off, L)] = corners[c]
            fx_v[pl.ds(j * L, L)] = fx
            fy_v[pl.ds(j * L, L)] = fy
            fz_v[pl.ds(j * L, L)] = fz
            return _

        lax.fori_loop(0, C // L, grp, None)

        copies = []
        for c in range(8):
            for s in range(NSUB):
                copies.append(pltpu.async_copy(
                    table_hbm.at[idx_v.at[c * NSUB + s]],
                    rows_v.at[c, pl.ds(s * 128, 128)],
                    sem,
                ))
        for cp in copies:
            cp.wait()

        def comb(j, _):
            fxg = fx_v[pl.ds(j * L, L)]
            fyg = fy_v[pl.ds(j * L, L)]
            fzg = fz_v[pl.ds(j * L, L)]
            for k in range(L):
                i = j * L + k
                fxk = lax.broadcast_in_dim(fxg[k], (L,), ())
                fyk = lax.broadcast_in_dim(fyg[k], (L,), ())
                fzk = lax.broadcast_in_dim(fzg[k], (L,), ())
                r000 = rows_v[0, i]
                r001 = rows_v[1, i]
                r010 = rows_v[2, i]
                r011 = rows_v[3, i]
                r100 = rows_v[4, i]
                r101 = rows_v[5, i]
                r110 = rows_v[6, i]
                r111 = rows_v[7, i]
                a00 = r000 + fxk * (r001 - r000)
                a01 = r010 + fxk * (r011 - r010)
                a10 = r100 + fxk * (r101 - r100)
                a11 = r110 + fxk * (r111 - r110)
                b0 = a00 + fyk * (a01 - a00)
                b1 = a10 + fyk * (a11 - a10)
                out_v[i] = b0 + fzk * (b1 - b0)
            return _

        lax.fori_loop(0, C // L, comb, None)

        pltpu.sync_copy(out_v, out_hbm.at[pl.ds(base, C)])
        return _

    lax.fori_loop(0, NCHUNK, chunk_body, None)


@functools.partial(
    pl.kernel,
    out_type=jax.ShapeDtypeStruct((B, F), jnp.float32),
    mesh=plsc.VectorSubcoreMesh(core_axis_name="c", subcore_axis_name="s"),
    compiler_params=_SC_PARAMS,
    scratch_types=[
        pltpu.VMEM((C * 3,), jnp.float32),
        pltpu.VMEM((8 * NSUB, 128), jnp.int32),
        pltpu.VMEM((C,), jnp.float32),
        pltpu.VMEM((C,), jnp.float32),
        pltpu.VMEM((C,), jnp.float32),
        pltpu.VMEM((8, C, F), jnp.float32),
        pltpu.VMEM((C, F), jnp.float32),
        pltpu.SemaphoreType.DMA,
    ],
)
def _sc_interp(x_hbm, table_hbm, out_hbm, coords_v, idx_v, fx_v, fy_v, fz_v,
               rows_v, out_v, sem):
    _body(x_hbm, table_hbm, out_hbm, coords_v, idx_v, fx_v, fy_v, fz_v,
          rows_v, out_v, sem)


def kernel(x, feature_grid):
    table = _sc_transpose(feature_grid.reshape(DHW * F))
    return _sc_interp(x.reshape(B * 3), table)
